# baseline (speedup 1.0000x reference)
import functools

import jax
import jax.numpy as jnp
from jax import lax
from jax.experimental import pallas as pl
from jax.experimental.pallas import tpu as pltpu
from jax.experimental.pallas import tpu_sc as plsc

BN = 4096
KSPLIT = 2


def _argmin_body(nb_total, k_total, x_ref, w_ref, idx_ref, wt_ref, w2_ref):
    f = pl.program_id(0)
    n = pl.program_id(1)

    @pl.when(n == 0)
    def _prep():
        wf = w_ref[0]
        wt_ref[0] = wf.T
        w2_ref[...] = jnp.sum(wf * wf, axis=0, keepdims=True)

    x = x_ref[0]
    x2 = jnp.sum(x * x, axis=1, keepdims=True)
    xs = x * -2.0
    ks = k_total // KSPLIT
    big = float(k_total)
    m_acc = i_acc = None
    for j in range(KSPLIT):
        w = w_ref[0, :, j * ks:(j + 1) * ks]
        w2 = w2_ref[:, j * ks:(j + 1) * ks]
        xw = jnp.dot(xs, w, preferred_element_type=jnp.float32)
        dist = x2 + xw + w2
        kk = (lax.broadcasted_iota(jnp.int32, dist.shape, 1)
              .astype(jnp.float32) + float(j * ks))
        if m_acc is None:
            m_acc, i_acc = dist, kk
        else:
            cond = dist < m_acc
            m_acc = jnp.where(cond, dist, m_acc)
            i_acc = jnp.where(cond, kk, i_acc)
    m = jnp.min(m_acc, axis=1, keepdims=True)
    idx_f = jnp.min(jnp.where(m_acc <= m, i_acc, big), axis=1, keepdims=True)
    idx_ref[0] = idx_f.astype(jnp.int32) + f * k_total


def _make_gather(b_total, d, n_chunks, chunk, nc, ns):
    b_per_w = n_chunks * chunk
    mesh = plsc.VectorSubcoreMesh(
        core_axis_name="c", subcore_axis_name="s", num_cores=nc, num_subcores=ns)

    @functools.partial(
        pl.kernel,
        mesh=mesh,
        out_type=jax.ShapeDtypeStruct((b_total, d), jnp.float32),
        scratch_types=[
            pltpu.VMEM((n_chunks, chunk), jnp.int32),
            pltpu.VMEM((chunk, d), jnp.float32),
            pltpu.VMEM((chunk, d), jnp.float32),
            pltpu.SemaphoreType.DMA,
            pltpu.SemaphoreType.DMA,
            pltpu.SemaphoreType.DMA,
        ],
    )
    def gather(table_hbm, idx_hbm, out_hbm, idx_v, buf0, buf1, semg, semo0, semo1):
        wid = lax.axis_index("s") * nc + lax.axis_index("c")
        base = wid * b_per_w
        pltpu.sync_copy(idx_hbm.at[wid], idx_v)
        bufs = (buf0, buf1)
        semos = (semo0, semo1)
        out_pend = [None, None]
        for c in range(n_chunks):
            b = bufs[c % 2]
            if out_pend[c % 2] is not None:
                out_pend[c % 2].wait()
            pltpu.async_copy(table_hbm.at[idx_v.at[c]], b, semg).wait()
            out_pend[c % 2] = pltpu.async_copy(
                b, out_hbm.at[pl.ds(base + c * chunk, chunk)], semos[c % 2])
        out_pend[(n_chunks - 1) % 2].wait()
        out_pend[n_chunks % 2].wait()

    return gather


def kernel(inputs, w):
    f_total, n_total, d = inputs.shape
    k_total = w.shape[2]
    nb = n_total // BN

    idx, wt = pl.pallas_call(
        functools.partial(_argmin_body, nb, k_total),
        grid=(f_total, nb),
        in_specs=[
            pl.BlockSpec((1, BN, d), lambda f, n: (f, n, 0)),
            pl.BlockSpec((1, d, k_total), lambda f, n: (f, 0, 0)),
        ],
        out_specs=[
            pl.BlockSpec((1, BN, 1), lambda f, n: (f * nb + n, 0, 0)),
            pl.BlockSpec((1, k_total, d), lambda f, n: (f, 0, 0)),
        ],
        out_shape=[
            jax.ShapeDtypeStruct((f_total * nb, BN, 1), jnp.int32),
            jax.ShapeDtypeStruct((f_total, k_total, d), jnp.float32),
        ],
        scratch_shapes=[pltpu.VMEM((1, k_total), jnp.float32)],
    )(inputs, w)

    nc, ns = 2, 16
    nw = nc * ns
    chunk = 128
    b_total = f_total * n_total
    n_chunks = b_total // (nw * chunk)

    idx3 = idx.reshape(nw, n_chunks, chunk)
    table = wt.reshape(f_total * k_total, d)
    out = _make_gather(b_total, d, n_chunks, chunk, nc, ns)(table, idx3)
    return out.reshape(f_total, n_total, d)


# speedup vs baseline: 1.0092x; 1.0092x over previous
import functools

import jax
import jax.numpy as jnp
from jax import lax
from jax.experimental import pallas as pl
from jax.experimental.pallas import tpu as pltpu
from jax.experimental.pallas import tpu_sc as plsc

BN = 2048
KSPLIT = 4


def _argmin_body(nb_total, k_total, x_ref, w_ref, idx_ref, wt_ref, w2_ref):
    f = pl.program_id(0)
    n = pl.program_id(1)

    @pl.when(n == 0)
    def _prep():
        wf = w_ref[0]
        wt_ref[0] = wf.T
        w2_ref[...] = jnp.sum(wf * wf, axis=0, keepdims=True)

    x = x_ref[0]
    x2 = jnp.sum(x * x, axis=1, keepdims=True)
    xs = x * -2.0
    ks = k_total // KSPLIT
    big = float(k_total)
    m_acc = i_acc = None
    for j in range(KSPLIT):
        w = w_ref[0, :, j * ks:(j + 1) * ks]
        w2 = w2_ref[:, j * ks:(j + 1) * ks]
        xw = jnp.dot(xs, w, preferred_element_type=jnp.float32)
        dist = x2 + xw + w2
        kk = (lax.broadcasted_iota(jnp.int32, dist.shape, 1)
              .astype(jnp.float32) + float(j * ks))
        if m_acc is None:
            m_acc, i_acc = dist, kk
        else:
            cond = dist < m_acc
            m_acc = jnp.where(cond, dist, m_acc)
            i_acc = jnp.where(cond, kk, i_acc)
    m = jnp.min(m_acc, axis=1, keepdims=True)
    idx_f = jnp.min(jnp.where(m_acc <= m, i_acc, big), axis=1, keepdims=True)
    idx_ref[0] = idx_f.astype(jnp.int32) + f * k_total


def _make_gather(b_total, d, n_chunks, chunk, nc, ns):
    b_per_w = n_chunks * chunk
    mesh = plsc.VectorSubcoreMesh(
        core_axis_name="c", subcore_axis_name="s", num_cores=nc, num_subcores=ns)

    @functools.partial(
        pl.kernel,
        mesh=mesh,
        out_type=jax.ShapeDtypeStruct((b_total, d), jnp.float32),
        scratch_types=[
            pltpu.VMEM((n_chunks, chunk), jnp.int32),
            pltpu.VMEM((chunk, d), jnp.float32),
            pltpu.VMEM((chunk, d), jnp.float32),
            pltpu.SemaphoreType.DMA,
            pltpu.SemaphoreType.DMA,
            pltpu.SemaphoreType.DMA,
        ],
    )
    def gather(table_hbm, idx_hbm, out_hbm, idx_v, buf0, buf1, semg, semo0, semo1):
        wid = lax.axis_index("s") * nc + lax.axis_index("c")
        base = wid * b_per_w
        pltpu.sync_copy(idx_hbm.at[wid], idx_v)
        bufs = (buf0, buf1)
        semos = (semo0, semo1)
        out_pend = [None, None]
        for c in range(n_chunks):
            b = bufs[c % 2]
            if out_pend[c % 2] is not None:
                out_pend[c % 2].wait()
            pltpu.async_copy(table_hbm.at[idx_v.at[c]], b, semg).wait()
            out_pend[c % 2] = pltpu.async_copy(
                b, out_hbm.at[pl.ds(base + c * chunk, chunk)], semos[c % 2])
        out_pend[(n_chunks - 1) % 2].wait()
        out_pend[n_chunks % 2].wait()

    return gather


def kernel(inputs, w):
    f_total, n_total, d = inputs.shape
    k_total = w.shape[2]
    nb = n_total // BN

    idx, wt = pl.pallas_call(
        functools.partial(_argmin_body, nb, k_total),
        grid=(f_total, nb),
        in_specs=[
            pl.BlockSpec((1, BN, d), lambda f, n: (f, n, 0)),
            pl.BlockSpec((1, d, k_total), lambda f, n: (f, 0, 0)),
        ],
        out_specs=[
            pl.BlockSpec((1, BN, 1), lambda f, n: (f * nb + n, 0, 0)),
            pl.BlockSpec((1, k_total, d), lambda f, n: (f, 0, 0)),
        ],
        out_shape=[
            jax.ShapeDtypeStruct((f_total * nb, BN, 1), jnp.int32),
            jax.ShapeDtypeStruct((f_total, k_total, d), jnp.float32),
        ],
        scratch_shapes=[pltpu.VMEM((1, k_total), jnp.float32)],
    )(inputs, w)

    nc, ns = 2, 16
    nw = nc * ns
    chunk = 128
    b_total = f_total * n_total
    n_chunks = b_total // (nw * chunk)

    idx3 = idx.reshape(nw, n_chunks, chunk)
    table = wt.reshape(f_total * k_total, d)
    out = _make_gather(b_total, d, n_chunks, chunk, nc, ns)(table, idx3)
    return out.reshape(f_total, n_total, d)


# speedup vs baseline: 1.0269x; 1.0176x over previous
import functools

import jax
import jax.numpy as jnp
from jax import lax
from jax.experimental import pallas as pl
from jax.experimental.pallas import tpu as pltpu
from jax.experimental.pallas import tpu_sc as plsc

BN = 4096
KSPLIT = 4


def _argmin_body(nb_total, k_total, x_ref, w_ref, idx_ref, wt_ref, w2_ref):
    f = pl.program_id(0)
    n = pl.program_id(1)

    @pl.when(n == 0)
    def _prep():
        wf = w_ref[0]
        wt_ref[0] = wf.T
        w2_ref[...] = jnp.sum(wf * wf, axis=0, keepdims=True)

    x = x_ref[0]
    x2 = jnp.sum(x * x, axis=1, keepdims=True)
    xs = x * -2.0
    ks = k_total // KSPLIT
    big = float(k_total)
    m_acc = i_acc = None
    for j in range(KSPLIT):
        w = w_ref[0, :, j * ks:(j + 1) * ks]
        w2 = w2_ref[:, j * ks:(j + 1) * ks]
        xw = jnp.dot(xs, w, preferred_element_type=jnp.float32)
        dist = x2 + xw + w2
        kk = (lax.broadcasted_iota(jnp.int32, dist.shape, 1)
              .astype(jnp.float32) + float(j * ks))
        if m_acc is None:
            m_acc, i_acc = dist, kk
        else:
            cond = dist < m_acc
            m_acc = jnp.where(cond, dist, m_acc)
            i_acc = jnp.where(cond, kk, i_acc)
    m = jnp.min(m_acc, axis=1, keepdims=True)
    idx_f = jnp.min(jnp.where(m_acc <= m, i_acc, big), axis=1, keepdims=True)
    idx_ref[0] = idx_f.astype(jnp.int32) + f * k_total


def _make_gather(b_total, d, n_chunks, chunk, nc, ns):
    b_per_w = n_chunks * chunk
    mesh = plsc.VectorSubcoreMesh(
        core_axis_name="c", subcore_axis_name="s", num_cores=nc, num_subcores=ns)

    @functools.partial(
        pl.kernel,
        mesh=mesh,
        out_type=jax.ShapeDtypeStruct((b_total, d), jnp.float32),
        scratch_types=[
            pltpu.VMEM((n_chunks, chunk), jnp.int32),
            pltpu.VMEM((chunk, d), jnp.float32),
            pltpu.VMEM((chunk, d), jnp.float32),
            pltpu.SemaphoreType.DMA,
            pltpu.SemaphoreType.DMA,
            pltpu.SemaphoreType.DMA,
        ],
    )
    def gather(table_hbm, idx_hbm, out_hbm, idx_v, buf0, buf1, semg, semo0, semo1):
        wid = lax.axis_index("s") * nc + lax.axis_index("c")
        base = wid * b_per_w
        pltpu.sync_copy(idx_hbm.at[wid], idx_v)
        bufs = (buf0, buf1)
        semos = (semo0, semo1)
        out_pend = [None, None]
        for c in range(n_chunks):
            b = bufs[c % 2]
            if out_pend[c % 2] is not None:
                out_pend[c % 2].wait()
            pltpu.async_copy(table_hbm.at[idx_v.at[c]], b, semg).wait()
            out_pend[c % 2] = pltpu.async_copy(
                b, out_hbm.at[pl.ds(base + c * chunk, chunk)], semos[c % 2])
        out_pend[(n_chunks - 1) % 2].wait()
        out_pend[n_chunks % 2].wait()

    return gather


def kernel(inputs, w):
    f_total, n_total, d = inputs.shape
    k_total = w.shape[2]
    nb = n_total // BN

    idx, wt = pl.pallas_call(
        functools.partial(_argmin_body, nb, k_total),
        grid=(f_total, nb),
        in_specs=[
            pl.BlockSpec((1, BN, d), lambda f, n: (f, n, 0)),
            pl.BlockSpec((1, d, k_total), lambda f, n: (f, 0, 0)),
        ],
        out_specs=[
            pl.BlockSpec((1, BN, 1), lambda f, n: (f * nb + n, 0, 0)),
            pl.BlockSpec((1, k_total, d), lambda f, n: (f, 0, 0)),
        ],
        out_shape=[
            jax.ShapeDtypeStruct((f_total * nb, BN, 1), jnp.int32),
            jax.ShapeDtypeStruct((f_total, k_total, d), jnp.float32),
        ],
        scratch_shapes=[pltpu.VMEM((1, k_total), jnp.float32)],
    )(inputs, w)

    nc, ns = 2, 16
    nw = nc * ns
    chunk = 128
    b_total = f_total * n_total
    n_chunks = b_total // (nw * chunk)

    idx3 = idx.reshape(nw, n_chunks, chunk)
    table = wt.reshape(f_total * k_total, d)
    out = _make_gather(b_total, d, n_chunks, chunk, nc, ns)(table, idx3)
    return out.reshape(f_total, n_total, d)
